# speedup vs baseline: 5.0761x; 5.0761x over previous
import functools

import jax
import jax.numpy as jnp
from jax import lax
from jax.experimental import pallas as pl
from jax.experimental.pallas import tpu as pltpu
from jax.experimental.pallas import tpu_sc as plsc

B = 16384
D = 128
NC, NS = 2, 16
NW = NC * NS
BPW = B // NW
TC_BLK = 2048


def _sc_gather_body(head_hbm, tail_hbm, rg_hbm, ap_hbm, ent_hbm, gn_hbm,
                    out_hbm, hidx, tidx, rows, sem):
    wid = lax.axis_index("s") * NC + lax.axis_index("c")
    base = wid * BPW
    pltpu.sync_copy(head_hbm.at[pl.ds(base, BPW)], hidx)
    pltpu.sync_copy(tail_hbm.at[pl.ds(base, BPW)], tidx)
    plan = [(rg_hbm, hidx, 0), (rg_hbm, tidx, 1),
            (ap_hbm, hidx, 2), (ap_hbm, tidx, 3),
            (ent_hbm, hidx, 4), (ent_hbm, tidx, 5),
            (gn_hbm, hidx, 6), (gn_hbm, tidx, 7)]
    for tab, idx, col in plan:
        pltpu.async_copy(tab.at[idx], rows, sem).wait()
        pltpu.sync_copy(rows, out_hbm.at[pl.ds(base, BPW),
                                         pl.ds(col * D, D)])


def _sc_gather(head, tail, rg, ap, ent, gn):
    mesh = plsc.VectorSubcoreMesh(core_axis_name="c", subcore_axis_name="s",
                                  num_cores=NC, num_subcores=NS)
    fn = functools.partial(
        pl.kernel, mesh=mesh,
        out_type=jax.ShapeDtypeStruct((B, 8 * D), jnp.float32),
        scratch_types=[
            pltpu.VMEM((BPW,), jnp.int32),
            pltpu.VMEM((BPW,), jnp.int32),
            pltpu.VMEM((BPW, D), jnp.float32),
            pltpu.SemaphoreType.DMA,
        ],
    )(_sc_gather_body)
    return fn(head, tail, rg, ap, ent, gn)


def _tc_mlp_body(rel_ref, g_ref, rel_emb_ref, w1a_ref, w1s_ref, w1g_ref,
                 b1_ref, w2_ref, b2_ref, out_ref):
    g = g_ref[...]
    rel = rel_ref[...]
    onehot = (rel == lax.broadcasted_iota(jnp.int32, (TC_BLK, 64), 1))
    r_e = jnp.dot(onehot.astype(jnp.float32), rel_emb_ref[...],
                  preferred_element_type=jnp.float32)
    s = jax.nn.sigmoid(g[:, 4 * D:5 * D] + r_e - g[:, 5 * D:6 * D])
    hid = (jnp.dot(g[:, :4 * D], w1a_ref[...],
                   preferred_element_type=jnp.float32)
           + jnp.dot(s, w1s_ref[...], preferred_element_type=jnp.float32)
           + jnp.dot(g[:, 6 * D:], w1g_ref[...],
                     preferred_element_type=jnp.float32)
           + b1_ref[...])
    hid = jnp.maximum(hid, 0.0)
    logits = jnp.dot(hid, w2_ref[...],
                     preferred_element_type=jnp.float32) + b2_ref[...]
    m = jnp.max(logits, axis=-1, keepdims=True)
    e = jnp.exp(logits - m)
    out_ref[...] = e / jnp.sum(e, axis=-1, keepdims=True)


def _tc_mlp(rel2d, g, rel_emb, w1a, w1s, w1g, b1, w2, b2):
    nblk = B // TC_BLK
    return pl.pallas_call(
        _tc_mlp_body,
        grid=(nblk,),
        in_specs=[
            pl.BlockSpec((TC_BLK, 1), lambda i: (i, 0)),
            pl.BlockSpec((TC_BLK, 8 * D), lambda i: (i, 0)),
            pl.BlockSpec((64, D), lambda i: (0, 0)),
            pl.BlockSpec((4 * D, D), lambda i: (0, 0)),
            pl.BlockSpec((D, D), lambda i: (0, 0)),
            pl.BlockSpec((2 * D, D), lambda i: (0, 0)),
            pl.BlockSpec((1, D), lambda i: (0, 0)),
            pl.BlockSpec((D, 2), lambda i: (0, 0)),
            pl.BlockSpec((1, 2), lambda i: (0, 0)),
        ],
        out_specs=pl.BlockSpec((TC_BLK, 2), lambda i: (i, 0)),
        out_shape=jax.ShapeDtypeStruct((B, 2), jnp.float32),
    )(rel2d, g, rel_emb, w1a, w1s, w1g, b1, w2, b2)


def kernel(head, rel, tail, ent_emb, rel_emb, rg_feature, ap_feature,
           gn_feature, W1, b1, W2, b2):
    g = _sc_gather(head.astype(jnp.int32), tail.astype(jnp.int32),
                   rg_feature, ap_feature, ent_emb, gn_feature)
    w1a = W1[:4 * D]
    w1s = W1[4 * D:5 * D]
    w1g = W1[5 * D:]
    return _tc_mlp(rel.astype(jnp.int32).reshape(B, 1), g, rel_emb,
                   w1a, w1s, w1g, b1.reshape(1, D), W2, b2.reshape(1, 2))
